# baseline (speedup 1.0000x reference)
import jax
import jax.numpy as jnp
from jax import lax
from jax.experimental import pallas as pl
from jax.experimental.pallas import tpu as pltpu
from jax.experimental.pallas import tpu_sc as plsc

NUM_EMBEDDINGS = 1000000
FEATURES = 32
PADF = 128
BATCH = 16384
SEQ = 20

NW = 32
BW = BATCH // NW
HALF = 256
NSTREAM = HALF // 128
FT = FEATURES // 8
NSTEP = SEQ * (BW // HALF)
NBUF = 2


def _body(idx_hbm, table_hbm, out_hbm, idx_v, rows_v, tiles_v, gsem, isem):
    wid = lax.axis_index("s") * 2 + lax.axis_index("c")
    b0 = wid * BW

    def idx_off(h):
        s = h // 2
        return s, b0 + (h % 2) * HALF

    def start_idx(h, b):
        s, o = idx_off(h)
        pltpu.async_copy(idx_hbm.at[s, pl.ds(o, HALF)], idx_v.at[b],
                         isem.at[b])

    def wait_idx(h, b):
        s, o = idx_off(h)
        pltpu.make_async_copy(idx_hbm.at[s, pl.ds(o, HALF)], idx_v.at[b],
                              isem.at[b]).wait()

    def start_gathers(b):
        for k in range(NSTREAM):
            pltpu.async_copy(table_hbm.at[idx_v.at[b, pl.ds(k * 128, 128)]],
                             rows_v.at[b, pl.ds(k * 128, 128)], gsem.at[b])

    def wait_gathers(b):
        for k in range(NSTREAM):
            pltpu.make_async_copy(
                table_hbm.at[idx_v.at[b, pl.ds(k * 128, 128)]],
                rows_v.at[b, pl.ds(k * 128, 128)], gsem.at[b]).wait()

    lanes = lax.iota(jnp.int32, 16)

    def transpose_and_write(h, b):
        @plsc.parallel_loop(0, FT * NSTREAM, unroll=2)
        def tc_step(tc):
            t = tc // NSTREAM
            c = tc % NSTREAM
            rows_c = lanes + c * 128
            for jm in range(8):
                j = jnp.broadcast_to(t * 8 + jm, (16,)).astype(jnp.int32)
                for g in range(8):
                    vals = plsc.load_gather(rows_v.at[b], [rows_c + g * 16, j])
                    tiles_v[t, c, jm, pl.ds(g * 16, 16)] = vals

        s = h // 2
        c0 = wid * (BW // 128) + (h % 2) * NSTREAM
        for t in range(FT):
            pltpu.sync_copy(tiles_v.at[t], out_hbm.at[s, t, pl.ds(c0, NSTREAM)])

    for b in range(NBUF):
        start_idx(b, b)
        wait_idx(b, b)
        start_gathers(b)

    def group(g, _):
        for b in range(NBUF):
            h = g * NBUF + b
            wait_gathers(b)
            start_idx(h + NBUF, b)
            transpose_and_write(h, b)
            wait_idx(h + NBUF, b)
            start_gathers(b)
        return ()

    lax.fori_loop(0, (NSTEP - NBUF) // NBUF, group, ())

    for b in range(NBUF):
        h = NSTEP - NBUF + b
        wait_gathers(b)
        transpose_and_write(h, b)


def kernel(inputs, embedding):
    idx_t = inputs.T
    table_p = jnp.pad(embedding, ((0, 0), (0, PADF - FEATURES)))
    mesh = plsc.VectorSubcoreMesh(core_axis_name="c", subcore_axis_name="s")
    out = pl.kernel(
        _body,
        mesh=mesh,
        out_type=jax.ShapeDtypeStruct((SEQ, FT, BATCH // 128, 8, 128),
                                      jnp.float32),
        scratch_types=[
            pltpu.VMEM((NBUF, HALF), jnp.int32),
            pltpu.VMEM((NBUF, HALF, PADF), jnp.float32),
            pltpu.VMEM((FT, NSTREAM, 8, 128), jnp.float32),
            pltpu.SemaphoreType.DMA((NBUF,)),
            pltpu.SemaphoreType.DMA((NBUF,)),
        ],
        compiler_params=pltpu.CompilerParams(use_tc_tiling_on_sc=False,
                                             needs_layout_passes=False),
    )(idx_t, table_p)
    return out.transpose(2, 4, 0, 1, 3).reshape(BATCH, SEQ, FEATURES)
